# baseline (speedup 1.0000x reference)
import functools

import jax
import jax.numpy as jnp
from jax import lax
from jax.experimental import pallas as pl
from jax.experimental.pallas import tpu as pltpu
from jax.experimental.pallas import tpu_sc as plsc

NVOCAB = 100000
NHID = 64
MAXLEN = 200
BATCH = 4096
SEQ = 200

NUM_CORES = 2
NUM_SUBCORES = 16
NW = NUM_CORES * NUM_SUBCORES
SEQ_PER_W = BATCH // NW
LANES = 16
NBUF = 6
LOOKAHEAD = 3

_mesh = plsc.VectorSubcoreMesh(core_axis_name="c", subcore_axis_name="s")


@functools.partial(
    pl.kernel,
    out_type=jax.ShapeDtypeStruct((BATCH, SEQ, NHID), jnp.float32),
    mesh=_mesh,
    scratch_types=[
        pltpu.VMEM((SEQ_PER_W, SEQ), jnp.int32),
        pltpu.VMEM((SEQ, NHID), jnp.float32),
        [pltpu.VMEM((SEQ, NHID), jnp.float32) for _ in range(NBUF)],
        [pltpu.SemaphoreType.DMA for _ in range(NBUF)],
        [pltpu.SemaphoreType.DMA for _ in range(NBUF)],
    ],
    compiler_params=pltpu.CompilerParams(use_tc_tiling_on_sc=False),
)
def _embed(x_hbm, emb_hbm, pos_hbm, out_hbm, idx_all, pos_v, rows, gsem, ssem):
    wid = lax.axis_index("s") * NUM_CORES + lax.axis_index("c")
    seq0 = wid * SEQ_PER_W

    pltpu.sync_copy(x_hbm.at[pl.ds(seq0, SEQ_PER_W)], idx_all)
    pltpu.sync_copy(pos_hbm, pos_v)

    def gather_desc(g, b):
        src = emb_hbm.at[idx_all.at[g]]
        return pltpu.make_async_copy(src, rows[b], gsem[b])

    def store_desc(g, b):
        return pltpu.make_async_copy(rows[b], out_hbm.at[seq0 + g], ssem[b])

    for b in range(LOOKAHEAD):
        gather_desc(b, b).start()

    def chunk_body(g, j, issue_next, wait_prev_store):
        nb = (j + LOOKAHEAD) % NBUF
        ng = g + LOOKAHEAD

        if issue_next:
            if wait_prev_store:
                store_desc(ng - NBUF, nb).wait()
            gather_desc(ng, nb).start()

        gather_desc(g, j).wait()

        def add_rows(r, c2, _rows=rows[j]):
            for c in range(NHID // LANES):
                sl = pl.ds(LANES * c, LANES)
                _rows[r, sl] += pos_v[r, sl]
            return c2

        lax.fori_loop(0, SEQ, add_rows, 0, unroll=8)
        store_desc(g, j).start()

    NSTEADY = (SEQ_PER_W - LOOKAHEAD) // NBUF * NBUF

    for g in range(NBUF):
        chunk_body(g, g, True, g >= NBUF - LOOKAHEAD)

    def step(t, carry):
        for j in range(NBUF):
            chunk_body(t * NBUF + j, j, True, True)
        return carry

    lax.fori_loop(1, NSTEADY // NBUF, step, 0)

    for g in range(NSTEADY, SEQ_PER_W):
        chunk_body(g, g % NBUF, g + LOOKAHEAD < SEQ_PER_W, True)

    for k in range(LOOKAHEAD):
        g = SEQ_PER_W - LOOKAHEAD + k
        store_desc(g, g % NBUF).wait()


def kernel(x, emb_table, pos_table):
    return _embed(x.astype(jnp.int32), emb_table, pos_table)


# speedup vs baseline: 1.2493x; 1.2493x over previous
import functools

import jax
import jax.numpy as jnp
from jax import lax
from jax.experimental import pallas as pl
from jax.experimental.pallas import tpu as pltpu
from jax.experimental.pallas import tpu_sc as plsc

NVOCAB = 100000
NHID = 64
MAXLEN = 200
BATCH = 4096
SEQ = 200

NUM_CORES = 2
NUM_SUBCORES = 16
NW = NUM_CORES * NUM_SUBCORES
SEQ_PER_W = BATCH // NW
CH = 2
ROWS_PER_CH = CH * SEQ
NCHUNK = SEQ_PER_W // CH
LANES = 16
NBUF = 4
LA_G = 2
LA_I = 3

_mesh = plsc.VectorSubcoreMesh(core_axis_name="c", subcore_axis_name="s")


@functools.partial(
    pl.kernel,
    out_type=jax.ShapeDtypeStruct((BATCH * SEQ, NHID), jnp.float32),
    mesh=_mesh,
    scratch_types=[
        pltpu.VMEM((SEQ, NHID), jnp.float32),
        [pltpu.VMEM((ROWS_PER_CH, NHID), jnp.float32) for _ in range(NBUF)],
        [pltpu.VMEM((ROWS_PER_CH,), jnp.int32) for _ in range(NBUF)],
        [pltpu.SemaphoreType.DMA for _ in range(NBUF)],
        [pltpu.SemaphoreType.DMA for _ in range(NBUF)],
        [pltpu.SemaphoreType.DMA for _ in range(NBUF)],
    ],
    compiler_params=pltpu.CompilerParams(use_tc_tiling_on_sc=False),
)
def _embed(x_hbm, emb_hbm, pos_hbm, out_hbm, pos_v, rows, idxs, gsem, ssem, isem):
    wid = lax.axis_index("s") * NUM_CORES + lax.axis_index("c")
    base = wid * (SEQ_PER_W * SEQ)

    pltpu.sync_copy(pos_hbm, pos_v)

    def idx_desc(g, b):
        return pltpu.make_async_copy(
            x_hbm.at[pl.ds(base + g * ROWS_PER_CH, ROWS_PER_CH)], idxs[b], isem[b])

    def gather_desc(g, b):
        return pltpu.make_async_copy(emb_hbm.at[idxs[b]], rows[b], gsem[b])

    def store_desc(g, b):
        return pltpu.make_async_copy(
            rows[b], out_hbm.at[pl.ds(base + g * ROWS_PER_CH, ROWS_PER_CH)], ssem[b])

    def chunk_body(g, j, issue_idx, wait_store, issue_gather):
        if issue_idx:
            idx_desc(g + LA_I, (j + LA_I) % NBUF).start()
        if wait_store:
            store_desc(g - (NBUF - LA_G), (j + LA_G) % NBUF).wait()
        if issue_gather:
            idx_desc(g + LA_G, (j + LA_G) % NBUF).wait()
            gather_desc(g + LA_G, (j + LA_G) % NBUF).start()

        gather_desc(g, j).wait()

        def add_rows(r, c2, _rows=rows[j]):
            for c in range(NHID // LANES):
                sl = pl.ds(LANES * c, LANES)
                p = pos_v[r, sl]
                _rows[r, sl] += p
                _rows[SEQ + r, sl] += p
            return c2

        lax.fori_loop(0, SEQ, add_rows, 0, unroll=4)
        store_desc(g, j).start()

    for n in range(LA_I):
        idx_desc(n, n % NBUF).start()
    for n in range(LA_G):
        idx_desc(n, n % NBUF).wait()
        gather_desc(n, n % NBUF).start()

    for g in range(NBUF):
        chunk_body(g, g, g + LA_I < NCHUNK, g >= NBUF - LA_G, g + LA_G < NCHUNK)

    NSTEADY = (NCHUNK - LA_I) // NBUF * NBUF

    def step(t, carry):
        for j in range(NBUF):
            chunk_body(t * NBUF + j, j, True, True, True)
        return carry

    lax.fori_loop(1, NSTEADY // NBUF, step, 0)

    for g in range(NSTEADY, NCHUNK):
        chunk_body(g, g % NBUF, g + LA_I < NCHUNK, True, g + LA_G < NCHUNK)

    for k in range(NBUF - LA_G):
        g = NCHUNK - (NBUF - LA_G) + k
        store_desc(g, g % NBUF).wait()


def kernel(x, emb_table, pos_table):
    xf = x.reshape(-1).astype(jnp.int32)
    out = _embed(xf, emb_table, pos_table)
    return out.reshape(BATCH, SEQ, NHID)


# speedup vs baseline: 1.3268x; 1.0620x over previous
import functools

import jax
import jax.numpy as jnp
from jax import lax
from jax.experimental import pallas as pl
from jax.experimental.pallas import tpu as pltpu
from jax.experimental.pallas import tpu_sc as plsc

NVOCAB = 100000
NHID = 64
MAXLEN = 200
BATCH = 4096
SEQ = 200

NUM_CORES = 2
NUM_SUBCORES = 16
NW = NUM_CORES * NUM_SUBCORES
CH = 2
ROWS_PER_CH = CH * SEQ
LANES = 16
NBUF = 4
LA_G = 2
LA_I = 3
NPIECE = 4

_mesh = plsc.VectorSubcoreMesh(core_axis_name="c", subcore_axis_name="s")


def _make_embed(nbatch):
    seq_per_w = nbatch // NW
    nchunk = seq_per_w // CH

    @functools.partial(
        pl.kernel,
        out_type=jax.ShapeDtypeStruct((nbatch * SEQ, NHID), jnp.float32),
        mesh=_mesh,
        scratch_types=[
            pltpu.VMEM((SEQ, NHID), jnp.float32),
            [pltpu.VMEM((ROWS_PER_CH, NHID), jnp.float32) for _ in range(NBUF)],
            [pltpu.VMEM((ROWS_PER_CH,), jnp.int32) for _ in range(NBUF)],
            [pltpu.SemaphoreType.DMA for _ in range(NBUF)],
            [pltpu.SemaphoreType.DMA for _ in range(NBUF)],
            [pltpu.SemaphoreType.DMA for _ in range(NBUF)],
        ],
        compiler_params=pltpu.CompilerParams(use_tc_tiling_on_sc=False),
    )
    def _embed(x_hbm, emb_hbm, pos_hbm, out_hbm, pos_v, rows, idxs, gsem, ssem, isem):
        wid = lax.axis_index("s") * NUM_CORES + lax.axis_index("c")
        base = wid * (seq_per_w * SEQ)

        pltpu.sync_copy(pos_hbm, pos_v)

        def idx_desc(g, b):
            return pltpu.make_async_copy(
                x_hbm.at[pl.ds(base + g * ROWS_PER_CH, ROWS_PER_CH)], idxs[b], isem[b])

        def gather_desc(g, b):
            return pltpu.make_async_copy(emb_hbm.at[idxs[b]], rows[b], gsem[b])

        def store_desc(g, b):
            return pltpu.make_async_copy(
                rows[b], out_hbm.at[pl.ds(base + g * ROWS_PER_CH, ROWS_PER_CH)], ssem[b])

        def chunk_body(g, j, issue_idx, wait_store, issue_gather):
            if issue_idx:
                idx_desc(g + LA_I, (j + LA_I) % NBUF).start()
            if wait_store:
                store_desc(g - (NBUF - LA_G), (j + LA_G) % NBUF).wait()
            if issue_gather:
                idx_desc(g + LA_G, (j + LA_G) % NBUF).wait()
                gather_desc(g + LA_G, (j + LA_G) % NBUF).start()

            gather_desc(g, j).wait()

            def add_rows(r, c2, _rows=rows[j]):
                for c in range(NHID // LANES):
                    sl = pl.ds(LANES * c, LANES)
                    p = pos_v[r, sl]
                    _rows[r, sl] += p
                    _rows[SEQ + r, sl] += p
                return c2

            lax.fori_loop(0, SEQ, add_rows, 0, unroll=4)
            store_desc(g, j).start()

        for n in range(LA_I):
            idx_desc(n, n % NBUF).start()
        for n in range(LA_G):
            idx_desc(n, n % NBUF).wait()
            gather_desc(n, n % NBUF).start()

        for g in range(NBUF):
            chunk_body(g, g, g + LA_I < nchunk, g >= NBUF - LA_G, g + LA_G < nchunk)

        nsteady = (nchunk - LA_I) // NBUF * NBUF

        def step(t, carry):
            for j in range(NBUF):
                chunk_body(t * NBUF + j, j, True, True, True)
            return carry

        lax.fori_loop(1, nsteady // NBUF, step, 0)

        for g in range(nsteady, nchunk):
            chunk_body(g, g % NBUF, g + LA_I < nchunk, True, g + LA_G < nchunk)

        for k in range(NBUF - LA_G):
            g = nchunk - (NBUF - LA_G) + k
            store_desc(g, g % NBUF).wait()

    return _embed


_embed_piece = _make_embed(BATCH // NPIECE)


def kernel(x, emb_table, pos_table):
    nb = BATCH // NPIECE
    pieces = []
    for k in range(NPIECE):
        xk = x[k * nb:(k + 1) * nb].reshape(-1).astype(jnp.int32)
        ok = _embed_piece(xk, emb_table, pos_table)
        pieces.append(ok.reshape(nb, SEQ, NHID))
    return jnp.concatenate(pieces, axis=0)
